# Initial kernel scaffold; baseline (speedup 1.0000x reference)
import jax
import jax.numpy as jnp
from jax.experimental import pallas as pl


def kernel(features, neighbors, W_agg, b_agg, W, b):
    raise NotImplementedError("write your pallas kernel here")


# speedup vs baseline: 1.3362x; 1.3362x over previous
import functools

import jax
import jax.numpy as jnp
from jax import lax
from jax.experimental import pallas as pl
from jax.experimental.pallas import tpu as pltpu
from jax.experimental.pallas import tpu_sc as plsc

N = 10000
K = 32
D = 128
NLANES = 16
NSEG = D // NLANES
NTILES = 32
NPAD = 10240
NPT = NPAD // NTILES
CN = 4
CE = CN * K
NCHUNKS = NPT // CN


def _aggsum_sc(idx_flat, features):
    mesh = plsc.VectorSubcoreMesh(core_axis_name="c", subcore_axis_name="s")

    @functools.partial(
        pl.kernel,
        out_type=jax.ShapeDtypeStruct((NPAD, D), jnp.float32),
        mesh=mesh,
        scratch_types=[
            pltpu.VMEM((CE,), jnp.int32),
            pltpu.VMEM((CE, D), jnp.float32),
            pltpu.VMEM((NPT, D), jnp.float32),
            pltpu.SemaphoreType.DMA,
        ],
    )
    def body(idx_hbm, feat_hbm, out_hbm, idx_v, rows_v, acc_v, sem):
        wid = lax.axis_index("s") * 2 + lax.axis_index("c")
        ebase = wid * (NPT * K)

        def chunk(ci, carry):
            pltpu.sync_copy(idx_hbm.at[pl.ds(ebase + ci * CE, CE)], idx_v)
            pltpu.async_copy(feat_hbm.at[idx_v], rows_v, sem).wait()
            for n in range(CN):
                r0 = n * K

                def kstep(k, accs, r0=r0):
                    return tuple(
                        accs[d] + rows_v[r0 + k, pl.ds(d * NLANES, NLANES)]
                        for d in range(NSEG)
                    )

                init = tuple(
                    rows_v[r0, pl.ds(d * NLANES, NLANES)] for d in range(NSEG)
                )
                accs = lax.fori_loop(1, K, kstep, init)
                row = ci * CN + n
                for d in range(NSEG):
                    acc_v[row, pl.ds(d * NLANES, NLANES)] = accs[d]
            return carry

        lax.fori_loop(0, NCHUNKS, chunk, 0)
        pltpu.sync_copy(acc_v, out_hbm.at[pl.ds(wid * NPT, NPT)])

    return body(idx_flat, features)


def _tc_body(feat, aggs, wa, ba, w1, w2, bb, out):
    mean = aggs[...] * (1.0 / K)
    a = jnp.dot(mean, wa[...], preferred_element_type=jnp.float32) + ba[...]
    a = jnp.maximum(a, 0.0)
    out[...] = (
        jnp.dot(feat[...], w1[...], preferred_element_type=jnp.float32)
        + jnp.dot(a, w2[...], preferred_element_type=jnp.float32)
        + bb[...]
    )


def _dense_tc(features, aggsum, Wa, ba, W1, W2, bb):
    BR = 1000
    return pl.pallas_call(
        _tc_body,
        grid=(N // BR,),
        in_specs=[
            pl.BlockSpec((BR, D), lambda i: (i, 0)),
            pl.BlockSpec((BR, D), lambda i: (i, 0)),
            pl.BlockSpec((D, D), lambda i: (0, 0)),
            pl.BlockSpec((1, D), lambda i: (0, 0)),
            pl.BlockSpec((D, D), lambda i: (0, 0)),
            pl.BlockSpec((D, D), lambda i: (0, 0)),
            pl.BlockSpec((1, D), lambda i: (0, 0)),
        ],
        out_specs=pl.BlockSpec((BR, D), lambda i: (i, 0)),
        out_shape=jax.ShapeDtypeStruct((N, D), jnp.float32),
    )(features, aggsum, Wa, ba, W1, W2, bb)


def kernel(features, neighbors, W_agg, b_agg, W, b):
    nbr = neighbors.astype(jnp.int32)
    nbr = jnp.pad(nbr, ((0, NPAD - N), (0, 0)))
    idx_flat = nbr.reshape(NPAD * K)
    aggsum = _aggsum_sc(idx_flat, features)
    return _dense_tc(
        features,
        aggsum[:N],
        W_agg,
        b_agg.reshape(1, D),
        W[:D],
        W[D:],
        b.reshape(1, D),
    )


# speedup vs baseline: 1.5676x; 1.1731x over previous
import functools

import jax
import jax.numpy as jnp
from jax import lax
from jax.experimental import pallas as pl
from jax.experimental.pallas import tpu as pltpu
from jax.experimental.pallas import tpu_sc as plsc

N = 10000
K = 32
D = 128
NLANES = 16
NSEG = D // NLANES
NTILES = 32
NPAD = 10240
NPT = NPAD // NTILES
EPT = NPT * K
CN = 4
CE = CN * K
NCHUNKS = NPT // CN


def _aggsum_sc(idx_flat, features):
    mesh = plsc.VectorSubcoreMesh(core_axis_name="c", subcore_axis_name="s")

    @functools.partial(
        pl.kernel,
        out_type=jax.ShapeDtypeStruct((NPAD, D), jnp.float32),
        mesh=mesh,
        scratch_types=[
            pltpu.VMEM((EPT,), jnp.int32),
            pltpu.VMEM((CE, D), jnp.float32),
            pltpu.VMEM((CE, D), jnp.float32),
            pltpu.VMEM((NPT, D), jnp.float32),
            pltpu.SemaphoreType.DMA,
            pltpu.SemaphoreType.DMA,
        ],
    )
    def body(idx_hbm, feat_hbm, out_hbm, idx_v, buf0, buf1, acc_v, sem0, sem1):
        wid = lax.axis_index("s") * 2 + lax.axis_index("c")
        pltpu.sync_copy(idx_hbm.at[pl.ds(wid * EPT, EPT)], idx_v)

        def start(ci, buf, sem):
            pltpu.async_copy(feat_hbm.at[idx_v.at[pl.ds(ci * CE, CE)]], buf, sem)

        def wait(buf, sem):
            pltpu.make_async_copy(feat_hbm.at[pl.ds(0, CE)], buf, sem).wait()

        def reduce_chunk(ci, buf):
            for n in range(CN):
                accs = [buf[n * K, pl.ds(d * NLANES, NLANES)] for d in range(NSEG)]
                for k in range(1, K):
                    for d in range(NSEG):
                        accs[d] = accs[d] + buf[n * K + k, pl.ds(d * NLANES, NLANES)]
                row = ci * CN + n
                for d in range(NSEG):
                    acc_v[row, pl.ds(d * NLANES, NLANES)] = accs[d]

        start(0, buf0, sem0)
        start(1, buf1, sem1)

        def pair(g, carry):
            wait(buf0, sem0)
            reduce_chunk(2 * g, buf0)

            @pl.when(g < NCHUNKS // 2 - 1)
            def _():
                start(2 * g + 2, buf0, sem0)

            wait(buf1, sem1)
            reduce_chunk(2 * g + 1, buf1)

            @pl.when(g < NCHUNKS // 2 - 1)
            def _():
                start(2 * g + 3, buf1, sem1)

            return carry

        lax.fori_loop(0, NCHUNKS // 2, pair, 0)
        pltpu.sync_copy(acc_v, out_hbm.at[pl.ds(wid * NPT, NPT)])

    return body(idx_flat, features)


def _tc_body(feat, aggs, wa, ba, w1, w2, bb, out):
    mean = aggs[...] * (1.0 / K)
    a = jnp.dot(mean, wa[...], preferred_element_type=jnp.float32) + ba[...]
    a = jnp.maximum(a, 0.0)
    out[...] = (
        jnp.dot(feat[...], w1[...], preferred_element_type=jnp.float32)
        + jnp.dot(a, w2[...], preferred_element_type=jnp.float32)
        + bb[...]
    )


def _dense_tc(features, aggsum, Wa, ba, W1, W2, bb):
    BR = 1000
    return pl.pallas_call(
        _tc_body,
        grid=(N // BR,),
        in_specs=[
            pl.BlockSpec((BR, D), lambda i: (i, 0)),
            pl.BlockSpec((BR, D), lambda i: (i, 0)),
            pl.BlockSpec((D, D), lambda i: (0, 0)),
            pl.BlockSpec((1, D), lambda i: (0, 0)),
            pl.BlockSpec((D, D), lambda i: (0, 0)),
            pl.BlockSpec((D, D), lambda i: (0, 0)),
            pl.BlockSpec((1, D), lambda i: (0, 0)),
        ],
        out_specs=pl.BlockSpec((BR, D), lambda i: (i, 0)),
        out_shape=jax.ShapeDtypeStruct((N, D), jnp.float32),
    )(features, aggsum, Wa, ba, W1, W2, bb)


def kernel(features, neighbors, W_agg, b_agg, W, b):
    nbr = neighbors.astype(jnp.int32)
    nbr = jnp.pad(nbr, ((0, NPAD - N), (0, 0)))
    idx_flat = nbr.reshape(NPAD * K)
    aggsum = _aggsum_sc(idx_flat, features)
    return _dense_tc(
        features,
        aggsum[:N],
        W_agg,
        b_agg.reshape(1, D),
        W[:D],
        W[D:],
        b.reshape(1, D),
    )


# speedup vs baseline: 5.5594x; 3.5465x over previous
import functools

import jax
import jax.numpy as jnp
from jax import lax
from jax.experimental import pallas as pl
from jax.experimental.pallas import tpu as pltpu
from jax.experimental.pallas import tpu_sc as plsc

N = 10000
K = 32
D = 128
NLANES = 16
NSEG = D // NLANES
NTILES = 32
NPAD = 10240
NPT = NPAD // NTILES
EPT = NPT * K
CN = 4
CE = CN * K
NCHUNKS = NPT // CN


def _aggsum_sc(idx_flat, features):
    mesh = plsc.VectorSubcoreMesh(core_axis_name="c", subcore_axis_name="s")

    @functools.partial(
        pl.kernel,
        out_type=pltpu.MemorySpace.HBM((NPAD, D), jnp.float32),
        mesh=mesh,
        scratch_types=[
            pltpu.VMEM((EPT,), jnp.int32),
            pltpu.VMEM((CE, D), jnp.float32),
            pltpu.VMEM((CE, D), jnp.float32),
            pltpu.VMEM((2 * CN, D), jnp.float32),
            pltpu.VMEM((2 * CN, D), jnp.float32),
            pltpu.VMEM_SHARED((N, D), jnp.float32),
            pltpu.SemaphoreType.DMA,
            pltpu.SemaphoreType.DMA,
            pltpu.SemaphoreType.DMA,
            pltpu.SemaphoreType.DMA,
        ],
    )
    def body(idx_hbm, feat_hbm, out_hbm, idx_v, buf0, buf1, ob0, ob1, tab,
             sem0, sem1, osem0, osem1):
        sid = lax.axis_index("s")
        cid = lax.axis_index("c")
        wid = cid * 16 + sid
        pltpu.sync_copy(feat_hbm.at[pl.ds(sid * 624, 624)], tab.at[pl.ds(sid * 624, 624)])

        @pl.when(sid == 0)
        def _():
            pltpu.sync_copy(feat_hbm.at[pl.ds(9984, 16)], tab.at[pl.ds(9984, 16)])
        pltpu.sync_copy(idx_hbm.at[pl.ds(wid * EPT, EPT)], idx_v)
        plsc.subcore_barrier()

        def start(ci, buf, sem):
            pltpu.async_copy(tab.at[idx_v.at[pl.ds(ci * CE, CE)]], buf, sem)

        def gwait(buf, sem):
            pltpu.make_async_copy(tab.at[pl.ds(0, CE)], buf, sem).wait()

        def owait(ob, osem):
            pltpu.make_async_copy(ob, out_hbm.at[pl.ds(0, 2 * CN)], osem).wait()

        def reduce_chunk(buf, ob, half):
            def nbody(n, c):
                accs = [buf[n * K, pl.ds(d * NLANES, NLANES)] for d in range(NSEG)]
                for k in range(1, K):
                    for d in range(NSEG):
                        accs[d] = accs[d] + buf[n * K + k, pl.ds(d * NLANES, NLANES)]
                for d in range(NSEG):
                    ob[half * CN + n, pl.ds(d * NLANES, NLANES)] = accs[d]
                return c

            lax.fori_loop(0, CN, nbody, 0)

        start(0, buf0, sem0)
        start(1, buf1, sem1)
        NQ = NCHUNKS // 4

        def quad(q, carry):
            @pl.when(q > 0)
            def _():
                owait(ob0, osem0)

            gwait(buf0, sem0)
            reduce_chunk(buf0, ob0, 0)
            start(4 * q + 2, buf0, sem0)
            gwait(buf1, sem1)
            reduce_chunk(buf1, ob0, 1)
            start(4 * q + 3, buf1, sem1)
            pltpu.async_copy(ob0, out_hbm.at[pl.ds(wid * NPT + q * 4 * CN, 2 * CN)], osem0)

            @pl.when(q > 0)
            def _():
                owait(ob1, osem1)

            gwait(buf0, sem0)
            reduce_chunk(buf0, ob1, 0)

            @pl.when(q < NQ - 1)
            def _():
                start(4 * q + 4, buf0, sem0)

            gwait(buf1, sem1)
            reduce_chunk(buf1, ob1, 1)

            @pl.when(q < NQ - 1)
            def _():
                start(4 * q + 5, buf1, sem1)

            pltpu.async_copy(ob1, out_hbm.at[pl.ds(wid * NPT + q * 4 * CN + 2 * CN, 2 * CN)], osem1)
            return carry

        lax.fori_loop(0, NQ, quad, 0)
        owait(ob0, osem0)
        owait(ob1, osem1)

    return body(idx_flat, features)


def _tc_body(feat, aggs, wa, ba, w1, w2, bb, out):
    mean = aggs[...] * (1.0 / K)
    a = jnp.dot(mean, wa[...], preferred_element_type=jnp.float32) + ba[...]
    a = jnp.maximum(a, 0.0)
    out[...] = (
        jnp.dot(feat[...], w1[...], preferred_element_type=jnp.float32)
        + jnp.dot(a, w2[...], preferred_element_type=jnp.float32)
        + bb[...]
    )


def _dense_tc(features, aggsum, Wa, ba, W1, W2, bb):
    BR = 1000
    return pl.pallas_call(
        _tc_body,
        grid=(N // BR,),
        in_specs=[
            pl.BlockSpec((BR, D), lambda i: (i, 0)),
            pl.BlockSpec((BR, D), lambda i: (i, 0)),
            pl.BlockSpec((D, D), lambda i: (0, 0)),
            pl.BlockSpec((1, D), lambda i: (0, 0)),
            pl.BlockSpec((D, D), lambda i: (0, 0)),
            pl.BlockSpec((D, D), lambda i: (0, 0)),
            pl.BlockSpec((1, D), lambda i: (0, 0)),
        ],
        out_specs=pl.BlockSpec((BR, D), lambda i: (i, 0)),
        out_shape=jax.ShapeDtypeStruct((N, D), jnp.float32),
    )(features, aggsum, Wa, ba, W1, W2, bb)


def kernel(features, neighbors, W_agg, b_agg, W, b):
    nbr = neighbors.astype(jnp.int32)
    nbr = jnp.pad(nbr, ((0, NPAD - N), (0, 0)))
    idx_flat = nbr.reshape(NPAD * K)
    aggsum = _aggsum_sc(idx_flat, features)
    return _dense_tc(
        features,
        aggsum[:N],
        W_agg,
        b_agg.reshape(1, D),
        W[:D],
        W[D:],
        b.reshape(1, D),
    )


# speedup vs baseline: 6.4712x; 1.1640x over previous
import functools

import numpy as np
import jax
import jax.numpy as jnp
from jax import lax
from jax.experimental import pallas as pl
from jax.experimental.pallas import tpu as pltpu
from jax.experimental.pallas import tpu_sc as plsc

N = 10000
K = 32
D = 128
NLANES = 16
NGRP = D // (2 * NLANES)
NTILES = 32
NPAD = 10240
NPT = NPAD // NTILES
EPT = NPT * K
CN = 4
CE = CN * K
NCHUNKS = NPT // CN

_PERM = np.concatenate(
    [np.concatenate([g * 32 + np.arange(0, 32, 2), g * 32 + np.arange(1, 32, 2)])
     for g in range(NGRP)]
)


def _aggsum_sc(idx_flat, feat_bf16):
    mesh = plsc.VectorSubcoreMesh(core_axis_name="c", subcore_axis_name="s")

    @functools.partial(
        pl.kernel,
        out_type=pltpu.MemorySpace.HBM((NPAD, D), jnp.float32),
        mesh=mesh,
        compiler_params=pltpu.CompilerParams(use_tc_tiling_on_sc=False),
        scratch_types=[
            pltpu.VMEM((EPT,), jnp.int32),
            pltpu.VMEM((CE, D // 2), jnp.int32),
            pltpu.VMEM((CE, D // 2), jnp.int32),
            pltpu.VMEM((2 * CN, D), jnp.float32),
            pltpu.VMEM((2 * CN, D), jnp.float32),
            pltpu.VMEM_SHARED((N, D // 2), jnp.int32),
            pltpu.SemaphoreType.DMA,
            pltpu.SemaphoreType.DMA,
            pltpu.SemaphoreType.DMA,
            pltpu.SemaphoreType.DMA,
        ],
    )
    def body(idx_hbm, feat_hbm, out_hbm, idx_v, buf0, buf1, ob0, ob1, tab,
             sem0, sem1, osem0, osem1):
        sid = lax.axis_index("s")
        cid = lax.axis_index("c")
        wid = cid * 16 + sid
        pltpu.sync_copy(feat_hbm.at[pl.ds(sid * 624, 624)], tab.at[pl.ds(sid * 624, 624)])

        @pl.when(sid == 0)
        def _():
            pltpu.sync_copy(feat_hbm.at[pl.ds(9984, 16)], tab.at[pl.ds(9984, 16)])
        pltpu.sync_copy(idx_hbm.at[pl.ds(wid * EPT, EPT)], idx_v)
        plsc.subcore_barrier()

        def start(ci, buf, sem):
            pltpu.async_copy(tab.at[idx_v.at[pl.ds(ci * CE, CE)]], buf, sem)

        def gwait(buf, sem):
            pltpu.make_async_copy(tab.at[pl.ds(0, CE)], buf, sem).wait()

        def owait(ob, osem):
            pltpu.make_async_copy(ob, out_hbm.at[pl.ds(0, 2 * CN)], osem).wait()

        def reduce_chunk(buf, ob, half):
            def nbody(n, c):
                def halves(r, g):
                    w = buf[r, pl.ds(g * NLANES, NLANES)]
                    lo = jax.lax.bitcast_convert_type(w << 16, jnp.float32)
                    hi = jax.lax.bitcast_convert_type(w & jnp.int32(-65536), jnp.float32)
                    return lo, hi

                accs = []
                for g in range(NGRP):
                    a, b = halves(n * K, g)
                    accs.extend([a, b])
                for k in range(1, K):
                    for g in range(NGRP):
                        a, b = halves(n * K + k, g)
                        accs[2 * g] = accs[2 * g] + a
                        accs[2 * g + 1] = accs[2 * g + 1] + b
                for d in range(2 * NGRP):
                    ob[half * CN + n, pl.ds(d * NLANES, NLANES)] = accs[d]
                return c

            lax.fori_loop(0, CN, nbody, 0)

        start(0, buf0, sem0)
        start(1, buf1, sem1)
        NQ = NCHUNKS // 4

        def quad(q, carry):
            @pl.when(q > 0)
            def _():
                owait(ob0, osem0)

            gwait(buf0, sem0)
            reduce_chunk(buf0, ob0, 0)
            start(4 * q + 2, buf0, sem0)
            gwait(buf1, sem1)
            reduce_chunk(buf1, ob0, 1)
            start(4 * q + 3, buf1, sem1)
            pltpu.async_copy(ob0, out_hbm.at[pl.ds(wid * NPT + q * 4 * CN, 2 * CN)], osem0)

            @pl.when(q > 0)
            def _():
                owait(ob1, osem1)

            gwait(buf0, sem0)
            reduce_chunk(buf0, ob1, 0)

            @pl.when(q < NQ - 1)
            def _():
                start(4 * q + 4, buf0, sem0)

            gwait(buf1, sem1)
            reduce_chunk(buf1, ob1, 1)

            @pl.when(q < NQ - 1)
            def _():
                start(4 * q + 5, buf1, sem1)

            pltpu.async_copy(ob1, out_hbm.at[pl.ds(wid * NPT + q * 4 * CN + 2 * CN, 2 * CN)], osem1)
            return carry

        lax.fori_loop(0, NQ, quad, 0)
        owait(ob0, osem0)
        owait(ob1, osem1)

    return body(idx_flat, feat_bf16)


def _tc_body(feat, aggs, wa, ba, w1, w2, bb, out):
    mean = aggs[...] * (1.0 / K)
    a = jnp.dot(mean, wa[...], preferred_element_type=jnp.float32) + ba[...]
    a = jnp.maximum(a, 0.0)
    out[...] = (
        jnp.dot(feat[...], w1[...], preferred_element_type=jnp.float32)
        + jnp.dot(a, w2[...], preferred_element_type=jnp.float32)
        + bb[...]
    )


def _dense_tc(features, aggsum, Wa, ba, W1, W2, bb):
    BR = 1000
    return pl.pallas_call(
        _tc_body,
        grid=(N // BR,),
        in_specs=[
            pl.BlockSpec((BR, D), lambda i: (i, 0)),
            pl.BlockSpec((BR, D), lambda i: (i, 0)),
            pl.BlockSpec((D, D), lambda i: (0, 0)),
            pl.BlockSpec((1, D), lambda i: (0, 0)),
            pl.BlockSpec((D, D), lambda i: (0, 0)),
            pl.BlockSpec((D, D), lambda i: (0, 0)),
            pl.BlockSpec((1, D), lambda i: (0, 0)),
        ],
        out_specs=pl.BlockSpec((BR, D), lambda i: (i, 0)),
        out_shape=jax.ShapeDtypeStruct((N, D), jnp.float32),
    )(features, aggsum, Wa, ba, W1, W2, bb)


def kernel(features, neighbors, W_agg, b_agg, W, b):
    nbr = neighbors.astype(jnp.int32)
    nbr = jnp.pad(nbr, ((0, NPAD - N), (0, 0)))
    idx_flat = nbr.reshape(NPAD * K)
    feat_packed = jax.lax.bitcast_convert_type(
        features.astype(jnp.bfloat16).reshape(N, D // 2, 2), jnp.int32
    )
    aggsum = _aggsum_sc(idx_flat, feat_packed)
    return _dense_tc(
        features,
        aggsum[:N],
        W_agg[_PERM, :],
        b_agg.reshape(1, D),
        W[:D],
        W[D:],
        b.reshape(1, D),
    )


# speedup vs baseline: 8.8623x; 1.3695x over previous
import functools

import jax
import jax.numpy as jnp
from jax import lax
from jax.experimental import pallas as pl
from jax.experimental.pallas import tpu as pltpu
from jax.experimental.pallas import tpu_sc as plsc

N = 10000
K = 32
D = 128
DW = D // 2
NLANES = 16
NGRP = DW // NLANES
NTILES = 32
NPT = 320
CN = 4
CE = CN * K


def _pack_body(feat, out):
    x = feat[...]
    lo = x[:, :DW].astype(jnp.bfloat16).astype(jnp.float32)
    hi = x[:, DW:].astype(jnp.bfloat16).astype(jnp.float32)
    lo_u = jax.lax.bitcast_convert_type(lo, jnp.uint32) >> 16
    hi_u = jax.lax.bitcast_convert_type(hi, jnp.uint32) & jnp.uint32(0xFFFF0000)
    out[...] = jax.lax.bitcast_convert_type(lo_u | hi_u, jnp.int32)


def _pack_tc(features):
    BR = 2000
    return pl.pallas_call(
        _pack_body,
        grid=(N // BR,),
        in_specs=[pl.BlockSpec((BR, D), lambda i: (i, 0))],
        out_specs=pl.BlockSpec((BR, DW), lambda i: (i, 0)),
        out_shape=jax.ShapeDtypeStruct((N, DW), jnp.int32),
    )(features)


def _aggsum_sc(neighbors_i32, feat_packed):
    mesh = plsc.VectorSubcoreMesh(core_axis_name="c", subcore_axis_name="s")

    @functools.partial(
        pl.kernel,
        out_type=pltpu.MemorySpace.HBM((N, D), jnp.float32),
        mesh=mesh,
        compiler_params=pltpu.CompilerParams(use_tc_tiling_on_sc=False),
        scratch_types=[
            pltpu.VMEM((NPT * K,), jnp.int32),
            pltpu.VMEM((CE, DW), jnp.int32),
            pltpu.VMEM((CE, DW), jnp.int32),
            pltpu.VMEM((2 * CN, D), jnp.float32),
            pltpu.VMEM((2 * CN, D), jnp.float32),
            pltpu.VMEM_SHARED((N, DW), jnp.int32),
            pltpu.SemaphoreType.DMA,
            pltpu.SemaphoreType.DMA,
            pltpu.SemaphoreType.DMA,
            pltpu.SemaphoreType.DMA,
        ],
    )
    def body(nbr_hbm, feat_hbm, out_hbm, idx_v, buf0, buf1, ob0, ob1, tab,
             sem0, sem1, osem0, osem1):
        sid = lax.axis_index("s")
        cid = lax.axis_index("c")
        wid = cid * 16 + sid
        base = wid * NPT
        nq = jnp.where(wid == NTILES - 1, 5, NPT // CN // 4)
        pltpu.sync_copy(feat_hbm.at[pl.ds(sid * 624, 624)], tab.at[pl.ds(sid * 624, 624)])

        @pl.when(sid == 0)
        def _():
            pltpu.sync_copy(feat_hbm.at[pl.ds(9984, 16)], tab.at[pl.ds(9984, 16)])

        @pl.when(wid < NTILES - 1)
        def _():
            pltpu.sync_copy(nbr_hbm.at[pl.ds(base * K, NPT * K)],
                            idx_v.at[pl.ds(0, NPT * K)])

        @pl.when(wid == NTILES - 1)
        def _():
            pltpu.sync_copy(nbr_hbm.at[pl.ds(base * K, 80 * K)],
                            idx_v.at[pl.ds(0, 80 * K)])
        plsc.subcore_barrier()

        lastc = nq * 4 - 1

        def start(ci, buf, sem):
            ci = jnp.minimum(ci, lastc)
            pltpu.async_copy(tab.at[idx_v.at[pl.ds(ci * CE, CE)]], buf, sem)

        def gwait(buf, sem):
            pltpu.make_async_copy(tab.at[pl.ds(0, CE)], buf, sem).wait()

        def owait(ob, osem):
            pltpu.make_async_copy(ob, out_hbm.at[pl.ds(0, 2 * CN)], osem).wait()

        def reduce_chunk(buf, ob, half):
            def nbody(n, c):
                def halves(r, g):
                    w = buf[r, pl.ds(g * NLANES, NLANES)]
                    lo = jax.lax.bitcast_convert_type(w << 16, jnp.float32)
                    hi = jax.lax.bitcast_convert_type(w & jnp.int32(-65536), jnp.float32)
                    return lo, hi

                accs = []
                for g in range(NGRP):
                    a, b = halves(n * K, g)
                    accs.extend([a, b])
                for k in range(1, K):
                    for g in range(NGRP):
                        a, b = halves(n * K + k, g)
                        accs[2 * g] = accs[2 * g] + a
                        accs[2 * g + 1] = accs[2 * g + 1] + b
                row = half * CN + n
                for g in range(NGRP):
                    ob[row, pl.ds(g * NLANES, NLANES)] = accs[2 * g]
                    ob[row, pl.ds(DW + g * NLANES, NLANES)] = accs[2 * g + 1]
                return c

            lax.fori_loop(0, CN, nbody, 0)

        start(0, buf0, sem0)
        start(1, buf1, sem1)

        def quad(q, carry):
            @pl.when(q > 0)
            def _():
                owait(ob0, osem0)

            gwait(buf0, sem0)
            reduce_chunk(buf0, ob0, 0)
            start(4 * q + 2, buf0, sem0)
            gwait(buf1, sem1)
            reduce_chunk(buf1, ob0, 1)
            start(4 * q + 3, buf1, sem1)
            pltpu.async_copy(ob0, out_hbm.at[pl.ds(base + q * 4 * CN, 2 * CN)], osem0)

            @pl.when(q > 0)
            def _():
                owait(ob1, osem1)

            gwait(buf0, sem0)
            reduce_chunk(buf0, ob1, 0)

            @pl.when(q < nq - 1)
            def _():
                start(4 * q + 4, buf0, sem0)

            gwait(buf1, sem1)
            reduce_chunk(buf1, ob1, 1)

            @pl.when(q < nq - 1)
            def _():
                start(4 * q + 5, buf1, sem1)

            pltpu.async_copy(ob1, out_hbm.at[pl.ds(base + q * 4 * CN + 2 * CN, 2 * CN)], osem1)
            return carry

        lax.fori_loop(0, nq, quad, 0)
        owait(ob0, osem0)
        owait(ob1, osem1)

    return body(neighbors_i32, feat_packed)


def _lin1_body(feat, w1, bb, out):
    out[...] = (
        jnp.dot(feat[...], w1[...], preferred_element_type=jnp.float32) + bb[...]
    )


def _lin1_tc(features, W1, bb):
    BR = 2000
    return pl.pallas_call(
        _lin1_body,
        grid=(N // BR,),
        in_specs=[
            pl.BlockSpec((BR, D), lambda i: (i, 0)),
            pl.BlockSpec((D, D), lambda i: (0, 0)),
            pl.BlockSpec((1, D), lambda i: (0, 0)),
        ],
        out_specs=pl.BlockSpec((BR, D), lambda i: (i, 0)),
        out_shape=jax.ShapeDtypeStruct((N, D), jnp.float32),
    )(features, W1, bb)


def _lin2_body(part, aggs, wa, ba, w2, out):
    mean = aggs[...] * (1.0 / K)
    a = jnp.dot(mean, wa[...], preferred_element_type=jnp.float32) + ba[...]
    a = jnp.maximum(a, 0.0)
    out[...] = part[...] + jnp.dot(a, w2[...], preferred_element_type=jnp.float32)


def _lin2_tc(partial, aggsum, Wa, ba, W2):
    BR = 2000
    return pl.pallas_call(
        _lin2_body,
        grid=(N // BR,),
        in_specs=[
            pl.BlockSpec((BR, D), lambda i: (i, 0)),
            pl.BlockSpec((BR, D), lambda i: (i, 0)),
            pl.BlockSpec((D, D), lambda i: (0, 0)),
            pl.BlockSpec((1, D), lambda i: (0, 0)),
            pl.BlockSpec((D, D), lambda i: (0, 0)),
        ],
        out_specs=pl.BlockSpec((BR, D), lambda i: (i, 0)),
        out_shape=jax.ShapeDtypeStruct((N, D), jnp.float32),
    )(partial, aggsum, Wa, ba, W2)


def kernel(features, neighbors, W_agg, b_agg, W, b):
    nbr = neighbors.astype(jnp.int32).reshape(N * K)
    feat_packed = _pack_tc(features)
    aggsum = _aggsum_sc(nbr, feat_packed)
    partial = _lin1_tc(features, W[:D], b.reshape(1, D))
    return _lin2_tc(partial, aggsum, W_agg, b_agg.reshape(1, D), W[D:])


# speedup vs baseline: 9.9493x; 1.1226x over previous
import functools

import jax
import jax.numpy as jnp
from jax import lax
from jax.experimental import pallas as pl
from jax.experimental.pallas import tpu as pltpu
from jax.experimental.pallas import tpu_sc as plsc

N = 10000
K = 32
D = 128
DW = D // 2
NLANES = 16
NGRP = DW // NLANES
NTILES = 32
NPT = 320
CN = 4
CE = CN * K


def _pack_tc(features):
    u = jax.lax.bitcast_convert_type(features, jnp.uint32)
    packed = (u[:, :DW] >> 16) | (u[:, DW:] & jnp.uint32(0xFFFF0000))
    return jax.lax.bitcast_convert_type(packed, jnp.int32)


def _aggsum_sc(neighbors_i32, feat_packed):
    mesh = plsc.VectorSubcoreMesh(core_axis_name="c", subcore_axis_name="s")

    @functools.partial(
        pl.kernel,
        out_type=pltpu.MemorySpace.HBM((N, D), jnp.float32),
        mesh=mesh,
        compiler_params=pltpu.CompilerParams(use_tc_tiling_on_sc=False),
        scratch_types=[
            pltpu.VMEM((NPT * K,), jnp.int32),
            pltpu.VMEM((CE, DW), jnp.int32),
            pltpu.VMEM((CE, DW), jnp.int32),
            pltpu.VMEM((2 * CN, D), jnp.float32),
            pltpu.VMEM((2 * CN, D), jnp.float32),
            pltpu.VMEM_SHARED((N, DW), jnp.int32),
            pltpu.SemaphoreType.DMA,
            pltpu.SemaphoreType.DMA,
            pltpu.SemaphoreType.DMA,
            pltpu.SemaphoreType.DMA,
        ],
    )
    def body(nbr_hbm, feat_hbm, out_hbm, idx_v, buf0, buf1, ob0, ob1, tab,
             sem0, sem1, osem0, osem1):
        sid = lax.axis_index("s")
        cid = lax.axis_index("c")
        wid = cid * 16 + sid
        base = wid * NPT
        nq = jnp.where(wid == NTILES - 1, 5, NPT // CN // 4)
        pltpu.sync_copy(feat_hbm.at[pl.ds(sid * 624, 624)], tab.at[pl.ds(sid * 624, 624)])

        @pl.when(sid == 0)
        def _():
            pltpu.sync_copy(feat_hbm.at[pl.ds(9984, 16)], tab.at[pl.ds(9984, 16)])

        @pl.when(wid < NTILES - 1)
        def _():
            pltpu.sync_copy(nbr_hbm.at[pl.ds(base * K, NPT * K)],
                            idx_v.at[pl.ds(0, NPT * K)])

        @pl.when(wid == NTILES - 1)
        def _():
            pltpu.sync_copy(nbr_hbm.at[pl.ds(base * K, 80 * K)],
                            idx_v.at[pl.ds(0, 80 * K)])
        plsc.subcore_barrier()

        lastc = nq * 4 - 1

        def start(ci, buf, sem):
            ci = jnp.minimum(ci, lastc)
            pltpu.async_copy(tab.at[idx_v.at[pl.ds(ci * CE, CE)]], buf, sem)

        def gwait(buf, sem):
            pltpu.make_async_copy(tab.at[pl.ds(0, CE)], buf, sem).wait()

        def owait(ob, osem):
            pltpu.make_async_copy(ob, out_hbm.at[pl.ds(0, 2 * CN)], osem).wait()

        def reduce_chunk(buf, ob, half):
            def nbody(n, c):
                def halves(r, g):
                    w = buf[r, pl.ds(g * NLANES, NLANES)]
                    lo = jax.lax.bitcast_convert_type(w << 16, jnp.float32)
                    hi = jax.lax.bitcast_convert_type(w, jnp.float32)
                    return lo, hi

                accs = []
                for g in range(NGRP):
                    a, b = halves(n * K, g)
                    accs.extend([a, b])
                for k in range(1, K):
                    for g in range(NGRP):
                        a, b = halves(n * K + k, g)
                        accs[2 * g] = accs[2 * g] + a
                        accs[2 * g + 1] = accs[2 * g + 1] + b
                row = half * CN + n
                for g in range(NGRP):
                    ob[row, pl.ds(g * NLANES, NLANES)] = accs[2 * g]
                    ob[row, pl.ds(DW + g * NLANES, NLANES)] = accs[2 * g + 1]
                return c

            lax.fori_loop(0, CN, nbody, 0)

        start(0, buf0, sem0)
        start(1, buf1, sem1)

        def quad(q, carry):
            @pl.when(q > 0)
            def _():
                owait(ob0, osem0)

            gwait(buf0, sem0)
            reduce_chunk(buf0, ob0, 0)
            start(4 * q + 2, buf0, sem0)
            gwait(buf1, sem1)
            reduce_chunk(buf1, ob0, 1)
            start(4 * q + 3, buf1, sem1)
            pltpu.async_copy(ob0, out_hbm.at[pl.ds(base + q * 4 * CN, 2 * CN)], osem0)

            @pl.when(q > 0)
            def _():
                owait(ob1, osem1)

            gwait(buf0, sem0)
            reduce_chunk(buf0, ob1, 0)

            @pl.when(q < nq - 1)
            def _():
                start(4 * q + 4, buf0, sem0)

            gwait(buf1, sem1)
            reduce_chunk(buf1, ob1, 1)

            @pl.when(q < nq - 1)
            def _():
                start(4 * q + 5, buf1, sem1)

            pltpu.async_copy(ob1, out_hbm.at[pl.ds(base + q * 4 * CN + 2 * CN, 2 * CN)], osem1)
            return carry

        lax.fori_loop(0, nq, quad, 0)
        owait(ob0, osem0)
        owait(ob1, osem1)

    return body(neighbors_i32, feat_packed)


def _lin1_body(feat, w1, bb, out):
    out[...] = (
        jnp.dot(feat[...], w1[...], preferred_element_type=jnp.float32) + bb[...]
    )


def _lin1_tc(features, W1, bb):
    BR = 2000
    return pl.pallas_call(
        _lin1_body,
        grid=(N // BR,),
        in_specs=[
            pl.BlockSpec((BR, D), lambda i: (i, 0)),
            pl.BlockSpec((D, D), lambda i: (0, 0)),
            pl.BlockSpec((1, D), lambda i: (0, 0)),
        ],
        out_specs=pl.BlockSpec((BR, D), lambda i: (i, 0)),
        out_shape=jax.ShapeDtypeStruct((N, D), jnp.float32),
    )(features, W1, bb)


def _lin2_body(part, aggs, wa, ba, w2, out):
    mean = aggs[...] * (1.0 / K)
    a = jnp.dot(mean, wa[...], preferred_element_type=jnp.float32) + ba[...]
    a = jnp.maximum(a, 0.0)
    out[...] = part[...] + jnp.dot(a, w2[...], preferred_element_type=jnp.float32)


def _lin2_tc(partial, aggsum, Wa, ba, W2):
    BR = 2000
    return pl.pallas_call(
        _lin2_body,
        grid=(N // BR,),
        in_specs=[
            pl.BlockSpec((BR, D), lambda i: (i, 0)),
            pl.BlockSpec((BR, D), lambda i: (i, 0)),
            pl.BlockSpec((D, D), lambda i: (0, 0)),
            pl.BlockSpec((1, D), lambda i: (0, 0)),
            pl.BlockSpec((D, D), lambda i: (0, 0)),
        ],
        out_specs=pl.BlockSpec((BR, D), lambda i: (i, 0)),
        out_shape=jax.ShapeDtypeStruct((N, D), jnp.float32),
    )(partial, aggsum, Wa, ba, W2)


def kernel(features, neighbors, W_agg, b_agg, W, b):
    nbr = neighbors.astype(jnp.int32).reshape(N * K)
    feat_packed = _pack_tc(features)
    aggsum = _aggsum_sc(nbr, feat_packed)
    partial = _lin1_tc(features, W[:D], b.reshape(1, D))
    return _lin2_tc(partial, aggsum, W_agg, b_agg.reshape(1, D), W[D:])
